# baseline (speedup 1.0000x reference)
import functools
import math

import jax
import jax.numpy as jnp
import numpy as np
from jax.experimental import pallas as pl

D_MODEL = 768
SEQ = 2048
IDX_HEADS = 4
IDX_DIM = 64
TOP_K = 256
HALF = D_MODEL // 2

SB1 = 512
QB = 256
NPROJ = 3 * D_MODEL + IDX_HEADS * 128 + 128 + 128

NIDX = IDX_HEADS * 128 + 128 + 128


def _proj_kernel(x_ref, w_ref, b_ref, cos_ref, sin_ref, y_ref, z_ref):
    x = x_ref[0]
    y = jnp.dot(x, w_ref[...], preferred_element_type=jnp.float32) + b_ref[...]
    cos = cos_ref[...]
    sin = sin_ref[...]
    q1 = y[:, 0:HALF]
    q2 = y[:, HALF:D_MODEL]
    k1 = y[:, D_MODEL:D_MODEL + HALF]
    k2 = y[:, D_MODEL + HALF:2 * D_MODEL]
    qr = jnp.concatenate([q1 * cos - q2 * sin, q1 * sin + q2 * cos], axis=1)
    kr = jnp.concatenate([k1 * cos - k2 * sin, k1 * sin + k2 * cos], axis=1)
    y_ref[0] = jnp.concatenate(
        [qr, kr, y[:, 2 * D_MODEL:3 * D_MODEL]], axis=1).astype(jnp.bfloat16)
    z_ref[0] = y[:, 3 * D_MODEL:]


def _radix_select_threshold(skey, k):
    imin = jnp.int32(-2147483648)
    prefix = jnp.zeros((skey.shape[0], 1), jnp.int32)
    for bit in range(31, -1, -1):
        bp = (1 << bit) if bit < 31 else -2147483648
        cand = prefix | jnp.int32(bp)
        scand = cand ^ imin
        cnt = jnp.count_nonzero(skey >= scand, axis=1, keepdims=True)
        prefix = jnp.where(cnt >= k, cand, prefix)
    return prefix ^ imin


def _attn_kernel(qi_ref, ki_ref, wi_ref, q_ref, k_ref, v_ref, wo_ref, bo_ref,
                 tri_ref, o_ref):
    qi = qi_ref[0]
    ki = ki_ref[0]
    wi = wi_ref[0]
    agg = jnp.zeros((QB, SEQ), jnp.float32)
    for h in range(IDX_HEADS):
        sh = jax.lax.dot_general(
            qi[:, h * 128:(h + 1) * 128], ki,
            (((1,), (1,)), ((), ())), preferred_element_type=jnp.float32)
        agg = agg + jnp.maximum(sh, 0.0) * wi[:, h:h + 1]

    q = q_ref[0]
    k = k_ref[0]
    logits = jax.lax.dot_general(
        q, k, (((1,), (1,)), ((), ())),
        preferred_element_type=jnp.float32) * (1.0 / math.sqrt(D_MODEL))

    b = jax.lax.bitcast_convert_type(agg, jnp.int32)
    skey = jnp.where(b >= 0, b, b ^ jnp.int32(0x7fffffff))
    skey = jnp.where(agg == 0.0, jnp.int32(0), skey)
    sthresh = _radix_select_threshold(skey, TOP_K)

    gt = skey > sthresh
    eq = skey == sthresh
    need = (TOP_K - jnp.sum(gt.astype(jnp.int32), axis=1, keepdims=True)
            ).astype(jnp.float32)
    eqf = eq.astype(jnp.float32)
    tri = tri_ref[...]
    offs = jnp.zeros((QB, 1), jnp.float32)
    parts = []
    for c in range(SEQ // 128):
        ch = eqf[:, c * 128:(c + 1) * 128]
        parts.append(jax.lax.dot_general(
            ch, tri, (((1,), (0,)), ((), ())),
            preferred_element_type=jnp.float32) + offs)
        offs = offs + jnp.sum(ch, axis=1, keepdims=True)
    cum = jnp.concatenate(parts, axis=1)
    sel = gt | (eq & (cum <= need))

    logits = jnp.where(sel, logits, -jnp.inf)
    m = jnp.max(logits, axis=1, keepdims=True)
    e = jnp.exp(logits - m)
    p = (e / jnp.sum(e, axis=1, keepdims=True)).astype(jnp.bfloat16)
    attn = jax.lax.dot_general(p, v_ref[0], (((1,), (0,)), ((), ())),
                               preferred_element_type=jnp.float32)
    out = jax.lax.dot_general(attn.astype(jnp.bfloat16),
                              wo_ref[...].astype(jnp.bfloat16),
                              (((1,), (1,)), ((), ())),
                              preferred_element_type=jnp.float32) + bo_ref[...]
    o_ref[0] = out


@jax.jit
def kernel(x, Wqkv, bqkv, Wq_idx, bq_idx, Wk_idx, bk_idx, Ww_idx, bw_idx, Wo,
           bo):
    B, S, D = x.shape

    wq_pad = jnp.zeros((IDX_HEADS * 128, D), jnp.float32)
    bq_pad = jnp.zeros((IDX_HEADS * 128,), jnp.float32)
    for h in range(IDX_HEADS):
        wq_pad = jax.lax.dynamic_update_slice(
            wq_pad, Wq_idx[h * IDX_DIM:(h + 1) * IDX_DIM], (h * 128, 0))
        bq_pad = jax.lax.dynamic_update_slice(
            bq_pad, bq_idx[h * IDX_DIM:(h + 1) * IDX_DIM], (h * 128,))
    wk_pad = jnp.zeros((128, D), jnp.float32).at[:IDX_DIM].set(Wk_idx)
    bk_pad = jnp.zeros((128,), jnp.float32).at[:IDX_DIM].set(bk_idx)
    ww_pad = jnp.zeros((128, D), jnp.float32).at[:IDX_HEADS].set(Ww_idx)
    bw_pad = jnp.zeros((128,), jnp.float32).at[:IDX_HEADS].set(bw_idx)
    w_all = jnp.concatenate([Wqkv, wq_pad, wk_pad, ww_pad], axis=0).T
    b_all = jnp.concatenate([bqkv, bq_pad, bk_pad, bw_pad])[None, :]

    inv_freq = 1.0 / (10000.0 ** (jnp.arange(HALF, dtype=jnp.float32) / HALF))
    t = jnp.arange(S, dtype=jnp.float32)
    freqs = jnp.outer(t, inv_freq)
    cos = jnp.cos(freqs)
    sin = jnp.sin(freqs)
    ii = jnp.arange(128, dtype=jnp.int32)
    tri = (ii[:, None] <= ii[None, :]).astype(jnp.float32)

    y, z = pl.pallas_call(
        _proj_kernel,
        grid=(B, S // SB1),
        in_specs=[
            pl.BlockSpec((1, SB1, D), lambda b, s: (b, s, 0)),
            pl.BlockSpec((D, NPROJ), lambda b, s: (0, 0)),
            pl.BlockSpec((1, NPROJ), lambda b, s: (0, 0)),
            pl.BlockSpec((SB1, HALF), lambda b, s: (s, 0)),
            pl.BlockSpec((SB1, HALF), lambda b, s: (s, 0)),
        ],
        out_specs=[
            pl.BlockSpec((1, SB1, 3 * D_MODEL), lambda b, s: (b, s, 0)),
            pl.BlockSpec((1, SB1, NIDX), lambda b, s: (b, s, 0)),
        ],
        out_shape=[
            jax.ShapeDtypeStruct((B, S, 3 * D_MODEL), jnp.bfloat16),
            jax.ShapeDtypeStruct((B, S, NIDX), jnp.float32),
        ],
    )(x, w_all, b_all, cos, sin)

    out = pl.pallas_call(
        _attn_kernel,
        grid=(B, S // QB),
        in_specs=[
            pl.BlockSpec((1, QB, IDX_HEADS * 128), lambda b, q: (b, q, 0)),
            pl.BlockSpec((1, SEQ, 128), lambda b, q: (b, 0, 4)),
            pl.BlockSpec((1, QB, 128), lambda b, q: (b, q, 5)),
            pl.BlockSpec((1, QB, D_MODEL), lambda b, q: (b, q, 0)),
            pl.BlockSpec((1, SEQ, D_MODEL), lambda b, q: (b, 0, 1)),
            pl.BlockSpec((1, SEQ, D_MODEL), lambda b, q: (b, 0, 2)),
            pl.BlockSpec((D_MODEL, D_MODEL), lambda b, q: (0, 0)),
            pl.BlockSpec((1, D_MODEL), lambda b, q: (0, 0)),
            pl.BlockSpec((128, 128), lambda b, q: (0, 0)),
        ],
        out_specs=pl.BlockSpec((1, QB, D_MODEL), lambda b, q: (b, q, 0)),
        out_shape=jax.ShapeDtypeStruct((B, S, D_MODEL), jnp.float32),
    )(z, z, z, y, y, y, Wo, bo[None, :], tri)
    return out


# speedup vs baseline: 1.0810x; 1.0810x over previous
import functools
import math

import jax
import jax.numpy as jnp
import numpy as np
from jax.experimental import pallas as pl

D_MODEL = 768
SEQ = 2048
IDX_HEADS = 4
IDX_DIM = 64
TOP_K = 256
HALF = D_MODEL // 2

SB1 = 512
QB = 256
NPROJ = 3 * D_MODEL + IDX_HEADS * 128 + 128 + 128

NIDX = IDX_HEADS * 128 + 128 + 128


def _proj_kernel(x_ref, wqkv_ref, bqkv_ref, widx_ref, bidx_ref, y_ref, z_ref):
    x = x_ref[0]
    y = jax.lax.dot_general(x, wqkv_ref[...], (((1,), (1,)), ((), ())),
                            preferred_element_type=jnp.float32) + bqkv_ref[...]
    z = jax.lax.dot_general(x, widx_ref[...], (((1,), (1,)), ((), ())),
                            preferred_element_type=jnp.float32) + bidx_ref[...]
    s_blk = pl.program_id(1)
    t = (jax.lax.broadcasted_iota(jnp.int32, (SB1, HALF), 0).astype(jnp.float32)
         + jnp.float32(SB1) * s_blk.astype(jnp.float32))
    j = jax.lax.broadcasted_iota(jnp.int32, (SB1, HALF), 1).astype(jnp.float32)
    freqs = t * jnp.exp(j * jnp.float32(-math.log(10000.0) / HALF))
    cos = jnp.cos(freqs)
    sin = jnp.sin(freqs)
    q1 = y[:, 0:HALF]
    q2 = y[:, HALF:D_MODEL]
    k1 = y[:, D_MODEL:D_MODEL + HALF]
    k2 = y[:, D_MODEL + HALF:2 * D_MODEL]
    qr = jnp.concatenate([q1 * cos - q2 * sin, q1 * sin + q2 * cos], axis=1)
    kr = jnp.concatenate([k1 * cos - k2 * sin, k1 * sin + k2 * cos], axis=1)
    y_ref[0] = jnp.concatenate(
        [qr, kr, y[:, 2 * D_MODEL:3 * D_MODEL]], axis=1).astype(jnp.bfloat16)
    z_ref[0] = z


def _radix_select_threshold(skey, k):
    imin = jnp.int32(-2147483648)
    prefix = jnp.zeros((skey.shape[0], 1), jnp.int32)
    for bit in range(31, -1, -1):
        bp = (1 << bit) if bit < 31 else -2147483648
        cand = prefix | jnp.int32(bp)
        scand = cand ^ imin
        cnt = jnp.count_nonzero(skey >= scand, axis=1, keepdims=True)
        prefix = jnp.where(cnt >= k, cand, prefix)
    return prefix ^ imin


def _attn_kernel(qi_ref, ki_ref, wi_ref, q_ref, k_ref, v_ref, wo_ref, bo_ref,
                 tri_ref, o_ref):
    qi = qi_ref[0]
    ki = ki_ref[0]
    wi = wi_ref[0]
    agg = jnp.zeros((QB, SEQ), jnp.float32)
    for h in range(IDX_HEADS):
        sh = jax.lax.dot_general(
            qi[:, h * 128:(h + 1) * 128], ki,
            (((1,), (1,)), ((), ())), preferred_element_type=jnp.float32)
        agg = agg + jnp.maximum(sh, 0.0) * wi[:, h:h + 1]

    q = q_ref[0]
    k = k_ref[0]
    logits = jax.lax.dot_general(
        q, k, (((1,), (1,)), ((), ())),
        preferred_element_type=jnp.float32) * (1.0 / math.sqrt(D_MODEL))

    b = jax.lax.bitcast_convert_type(agg, jnp.int32)
    skey = jnp.where(b >= 0, b, b ^ jnp.int32(0x7fffffff))
    skey = jnp.where(agg == 0.0, jnp.int32(0), skey)
    sthresh = _radix_select_threshold(skey, TOP_K)

    gt = skey > sthresh
    eq = skey == sthresh
    need = (TOP_K - jnp.sum(gt.astype(jnp.int32), axis=1, keepdims=True)
            ).astype(jnp.float32)
    eqf = eq.astype(jnp.float32)
    tri = tri_ref[...]
    offs = jnp.zeros((QB, 1), jnp.float32)
    parts = []
    for c in range(SEQ // 128):
        ch = eqf[:, c * 128:(c + 1) * 128]
        parts.append(jax.lax.dot_general(
            ch, tri, (((1,), (0,)), ((), ())),
            preferred_element_type=jnp.float32) + offs)
        offs = offs + jnp.sum(ch, axis=1, keepdims=True)
    cum = jnp.concatenate(parts, axis=1)
    sel = gt | (eq & (cum <= need))

    logits = jnp.where(sel, logits, -jnp.inf)
    m = jnp.max(logits, axis=1, keepdims=True)
    e = jnp.exp(logits - m)
    p = (e / jnp.sum(e, axis=1, keepdims=True)).astype(jnp.bfloat16)
    attn = jax.lax.dot_general(p, v_ref[0], (((1,), (0,)), ((), ())),
                               preferred_element_type=jnp.float32)
    out = jax.lax.dot_general(attn.astype(jnp.bfloat16),
                              wo_ref[...].astype(jnp.bfloat16),
                              (((1,), (1,)), ((), ())),
                              preferred_element_type=jnp.float32) + bo_ref[...]
    o_ref[0] = out


@jax.jit
def kernel(x, Wqkv, bqkv, Wq_idx, bq_idx, Wk_idx, bk_idx, Ww_idx, bw_idx, Wo,
           bo):
    B, S, D = x.shape

    w_idx = jnp.zeros((NIDX, D), jnp.float32)
    b_idx = jnp.zeros((NIDX,), jnp.float32)
    for h in range(IDX_HEADS):
        w_idx = jax.lax.dynamic_update_slice(
            w_idx, Wq_idx[h * IDX_DIM:(h + 1) * IDX_DIM], (h * 128, 0))
        b_idx = jax.lax.dynamic_update_slice(
            b_idx, bq_idx[h * IDX_DIM:(h + 1) * IDX_DIM], (h * 128,))
    w_idx = jax.lax.dynamic_update_slice(w_idx, Wk_idx, (IDX_HEADS * 128, 0))
    b_idx = jax.lax.dynamic_update_slice(b_idx, bk_idx, (IDX_HEADS * 128,))
    w_idx = jax.lax.dynamic_update_slice(w_idx, Ww_idx, (IDX_HEADS * 128 + 128, 0))
    b_idx = jax.lax.dynamic_update_slice(b_idx, bw_idx, (IDX_HEADS * 128 + 128,))

    ii = jnp.arange(128, dtype=jnp.int32)
    tri = (ii[:, None] <= ii[None, :]).astype(jnp.float32)

    y, z = pl.pallas_call(
        _proj_kernel,
        grid=(B, S // SB1),
        in_specs=[
            pl.BlockSpec((1, SB1, D), lambda b, s: (b, s, 0)),
            pl.BlockSpec((3 * D_MODEL, D), lambda b, s: (0, 0)),
            pl.BlockSpec((1, 3 * D_MODEL), lambda b, s: (0, 0)),
            pl.BlockSpec((NIDX, D), lambda b, s: (0, 0)),
            pl.BlockSpec((1, NIDX), lambda b, s: (0, 0)),
        ],
        out_specs=[
            pl.BlockSpec((1, SB1, 3 * D_MODEL), lambda b, s: (b, s, 0)),
            pl.BlockSpec((1, SB1, NIDX), lambda b, s: (b, s, 0)),
        ],
        out_shape=[
            jax.ShapeDtypeStruct((B, S, 3 * D_MODEL), jnp.bfloat16),
            jax.ShapeDtypeStruct((B, S, NIDX), jnp.float32),
        ],
    )(x, Wqkv, bqkv[None, :], w_idx, b_idx[None, :])

    out = pl.pallas_call(
        _attn_kernel,
        grid=(B, S // QB),
        in_specs=[
            pl.BlockSpec((1, QB, IDX_HEADS * 128), lambda b, q: (b, q, 0)),
            pl.BlockSpec((1, SEQ, 128), lambda b, q: (b, 0, 4)),
            pl.BlockSpec((1, QB, 128), lambda b, q: (b, q, 5)),
            pl.BlockSpec((1, QB, D_MODEL), lambda b, q: (b, q, 0)),
            pl.BlockSpec((1, SEQ, D_MODEL), lambda b, q: (b, 0, 1)),
            pl.BlockSpec((1, SEQ, D_MODEL), lambda b, q: (b, 0, 2)),
            pl.BlockSpec((D_MODEL, D_MODEL), lambda b, q: (0, 0)),
            pl.BlockSpec((1, D_MODEL), lambda b, q: (0, 0)),
            pl.BlockSpec((128, 128), lambda b, q: (0, 0)),
        ],
        out_specs=pl.BlockSpec((1, QB, D_MODEL), lambda b, q: (b, q, 0)),
        out_shape=jax.ShapeDtypeStruct((B, S, D_MODEL), jnp.float32),
    )(z, z, z, y, y, y, Wo, bo[None, :], tri)
    return out


# speedup vs baseline: 1.1887x; 1.0997x over previous
import functools
import math

import jax
import jax.numpy as jnp
import numpy as np
from jax.experimental import pallas as pl

D_MODEL = 768
SEQ = 2048
IDX_HEADS = 4
IDX_DIM = 64
TOP_K = 256
HALF = D_MODEL // 2

SB1 = 512
QB = 256
NPROJ = 3 * D_MODEL + IDX_HEADS * 128 + 128 + 128

NIDX = IDX_HEADS * 128 + 128 + 128


def _proj_kernel(x_ref, wqkv_ref, bqkv_ref, widx_ref, bidx_ref, y_ref, z_ref):
    x = x_ref[0]
    y = jax.lax.dot_general(x, wqkv_ref[...], (((1,), (1,)), ((), ())),
                            preferred_element_type=jnp.float32) + bqkv_ref[...]
    z = jax.lax.dot_general(x, widx_ref[...], (((1,), (1,)), ((), ())),
                            preferred_element_type=jnp.float32) + bidx_ref[...]
    s_blk = pl.program_id(1)
    t = (jax.lax.broadcasted_iota(jnp.int32, (SB1, HALF), 0).astype(jnp.float32)
         + jnp.float32(SB1) * s_blk.astype(jnp.float32))
    j = jax.lax.broadcasted_iota(jnp.int32, (SB1, HALF), 1).astype(jnp.float32)
    freqs = t * jnp.exp(j * jnp.float32(-math.log(10000.0) / HALF))
    cos = jnp.cos(freqs)
    sin = jnp.sin(freqs)
    q1 = y[:, 0:HALF]
    q2 = y[:, HALF:D_MODEL]
    k1 = y[:, D_MODEL:D_MODEL + HALF]
    k2 = y[:, D_MODEL + HALF:2 * D_MODEL]
    qr = jnp.concatenate([q1 * cos - q2 * sin, q1 * sin + q2 * cos], axis=1)
    kr = jnp.concatenate([k1 * cos - k2 * sin, k1 * sin + k2 * cos], axis=1)
    y_ref[0] = jnp.concatenate(
        [qr, kr, y[:, 2 * D_MODEL:3 * D_MODEL]], axis=1).astype(jnp.bfloat16)
    z_ref[0] = z


def _radix_select_threshold(skey, k):
    imin = jnp.int32(-2147483648)
    prefix = jnp.zeros((skey.shape[0], 1), jnp.int32)
    for bit in range(31, -1, -1):
        bp = (1 << bit) if bit < 31 else -2147483648
        cand = prefix | jnp.int32(bp)
        scand = cand ^ imin
        cnt = jnp.count_nonzero(skey >= scand, axis=1, keepdims=True)
        prefix = jnp.where(cnt >= k, cand, prefix)
    return prefix ^ imin


def _count_ge_rows(x16, scand16):
    m = jnp.where(x16 >= scand16, jnp.int16(1), jnp.int16(0))
    parts = [m[:, c * 128:(c + 1) * 128] for c in range(SEQ // 128)]
    while len(parts) > 1:
        parts = [parts[i] + parts[i + 1] for i in range(0, len(parts), 2)]
    return jnp.sum(parts[0].astype(jnp.int32), axis=1,
                   keepdims=True).astype(jnp.float32)


def _attn_kernel(qi_ref, ki_ref, wi_ref, q_ref, k_ref, v_ref, wo_ref, bo_ref,
                 tri_ref, o_ref):
    qi = qi_ref[0]
    ki = ki_ref[0]
    wi = wi_ref[0]
    agg = jnp.zeros((QB, SEQ), jnp.float32)
    for h in range(IDX_HEADS):
        sh = jax.lax.dot_general(
            qi[:, h * 128:(h + 1) * 128], ki,
            (((1,), (1,)), ((), ())), preferred_element_type=jnp.float32)
        agg = agg + jnp.maximum(sh, 0.0) * wi[:, h:h + 1]

    q = q_ref[0]
    k = k_ref[0]
    logits = jax.lax.dot_general(
        q, k, (((1,), (1,)), ((), ())),
        preferred_element_type=jnp.float32) * (1.0 / math.sqrt(D_MODEL))

    b = jax.lax.bitcast_convert_type(agg, jnp.int32)
    skey = jnp.where(b >= 0, b, b ^ jnp.int32(0x7fffffff))
    skey = jnp.where(agg == 0.0, jnp.int32(0), skey)

    sh = (skey >> 16).astype(jnp.int16)
    sl = (skey ^ jnp.int32(0x8000)).astype(jnp.int16)
    kf = jnp.float32(TOP_K)

    hpref = jnp.zeros((QB, 1), jnp.int32)
    for bit in range(15, -1, -1):
        cand = hpref | jnp.int32(1 << bit)
        scand = (cand ^ jnp.int32(0x8000)).astype(jnp.int16)
        cnt = _count_ge_rows(sh, scand)
        hpref = jnp.where(cnt >= kf, cand, hpref)
    sH = (hpref ^ jnp.int32(0x8000)).astype(jnp.int16)

    hi_gt = sh > sH
    g_cnt = _count_ge_rows(jnp.where(hi_gt, jnp.int16(1), jnp.int16(0)),
                           jnp.int16(1))
    k2f = kf - g_cnt
    active = sh == sH
    spl = jnp.where(active, sl, jnp.int16(-32768))

    lpref = jnp.zeros((QB, 1), jnp.int32)
    for bit in range(15, -1, -1):
        cand = lpref | jnp.int32(1 << bit)
        scand = (cand ^ jnp.int32(0x8000)).astype(jnp.int16)
        cnt = _count_ge_rows(spl, scand)
        lpref = jnp.where(cnt >= k2f, cand, lpref)
    sL = (lpref ^ jnp.int32(0x8000)).astype(jnp.int16)

    gt = hi_gt | (active & (spl > sL))
    eq = active & (spl == sL)
    need = kf - _count_ge_rows(jnp.where(gt, jnp.int16(1), jnp.int16(0)),
                               jnp.int16(1))
    eqf = eq.astype(jnp.float32)
    tri = tri_ref[...]
    offs = jnp.zeros((QB, 1), jnp.float32)
    parts = []
    for c in range(SEQ // 128):
        ch = eqf[:, c * 128:(c + 1) * 128]
        parts.append(jax.lax.dot_general(
            ch, tri, (((1,), (0,)), ((), ())),
            preferred_element_type=jnp.float32) + offs)
        offs = offs + jnp.sum(ch, axis=1, keepdims=True)
    cum = jnp.concatenate(parts, axis=1)
    sel = gt | (eq & (cum <= need))

    logits = jnp.where(sel, logits, -jnp.inf)
    m = jnp.max(logits, axis=1, keepdims=True)
    e = jnp.exp(logits - m)
    p = (e / jnp.sum(e, axis=1, keepdims=True)).astype(jnp.bfloat16)
    attn = jax.lax.dot_general(p, v_ref[0], (((1,), (0,)), ((), ())),
                               preferred_element_type=jnp.float32)
    out = jax.lax.dot_general(attn.astype(jnp.bfloat16),
                              wo_ref[...].astype(jnp.bfloat16),
                              (((1,), (1,)), ((), ())),
                              preferred_element_type=jnp.float32) + bo_ref[...]
    o_ref[0] = out


@jax.jit
def kernel(x, Wqkv, bqkv, Wq_idx, bq_idx, Wk_idx, bk_idx, Ww_idx, bw_idx, Wo,
           bo):
    B, S, D = x.shape

    w_idx = jnp.zeros((NIDX, D), jnp.float32)
    b_idx = jnp.zeros((NIDX,), jnp.float32)
    for h in range(IDX_HEADS):
        w_idx = jax.lax.dynamic_update_slice(
            w_idx, Wq_idx[h * IDX_DIM:(h + 1) * IDX_DIM], (h * 128, 0))
        b_idx = jax.lax.dynamic_update_slice(
            b_idx, bq_idx[h * IDX_DIM:(h + 1) * IDX_DIM], (h * 128,))
    w_idx = jax.lax.dynamic_update_slice(w_idx, Wk_idx, (IDX_HEADS * 128, 0))
    b_idx = jax.lax.dynamic_update_slice(b_idx, bk_idx, (IDX_HEADS * 128,))
    w_idx = jax.lax.dynamic_update_slice(w_idx, Ww_idx, (IDX_HEADS * 128 + 128, 0))
    b_idx = jax.lax.dynamic_update_slice(b_idx, bw_idx, (IDX_HEADS * 128 + 128,))

    ii = jnp.arange(128, dtype=jnp.int32)
    tri = (ii[:, None] <= ii[None, :]).astype(jnp.float32)

    y, z = pl.pallas_call(
        _proj_kernel,
        grid=(B, S // SB1),
        in_specs=[
            pl.BlockSpec((1, SB1, D), lambda b, s: (b, s, 0)),
            pl.BlockSpec((3 * D_MODEL, D), lambda b, s: (0, 0)),
            pl.BlockSpec((1, 3 * D_MODEL), lambda b, s: (0, 0)),
            pl.BlockSpec((NIDX, D), lambda b, s: (0, 0)),
            pl.BlockSpec((1, NIDX), lambda b, s: (0, 0)),
        ],
        out_specs=[
            pl.BlockSpec((1, SB1, 3 * D_MODEL), lambda b, s: (b, s, 0)),
            pl.BlockSpec((1, SB1, NIDX), lambda b, s: (b, s, 0)),
        ],
        out_shape=[
            jax.ShapeDtypeStruct((B, S, 3 * D_MODEL), jnp.bfloat16),
            jax.ShapeDtypeStruct((B, S, NIDX), jnp.float32),
        ],
    )(x, Wqkv, bqkv[None, :], w_idx, b_idx[None, :])

    out = pl.pallas_call(
        _attn_kernel,
        grid=(B, S // QB),
        in_specs=[
            pl.BlockSpec((1, QB, IDX_HEADS * 128), lambda b, q: (b, q, 0)),
            pl.BlockSpec((1, SEQ, 128), lambda b, q: (b, 0, 4)),
            pl.BlockSpec((1, QB, 128), lambda b, q: (b, q, 5)),
            pl.BlockSpec((1, QB, D_MODEL), lambda b, q: (b, q, 0)),
            pl.BlockSpec((1, SEQ, D_MODEL), lambda b, q: (b, 0, 1)),
            pl.BlockSpec((1, SEQ, D_MODEL), lambda b, q: (b, 0, 2)),
            pl.BlockSpec((D_MODEL, D_MODEL), lambda b, q: (0, 0)),
            pl.BlockSpec((1, D_MODEL), lambda b, q: (0, 0)),
            pl.BlockSpec((128, 128), lambda b, q: (0, 0)),
        ],
        out_specs=pl.BlockSpec((1, QB, D_MODEL), lambda b, q: (b, q, 0)),
        out_shape=jax.ShapeDtypeStruct((B, S, D_MODEL), jnp.float32),
    )(z, z, z, y, y, y, Wo, bo[None, :], tri)
    return out


# speedup vs baseline: 1.2218x; 1.0278x over previous
import functools
import math

import jax
import jax.numpy as jnp
import numpy as np
from jax.experimental import pallas as pl

D_MODEL = 768
SEQ = 2048
IDX_HEADS = 4
IDX_DIM = 64
TOP_K = 256
HALF = D_MODEL // 2

SB1 = 512
QB = 512
NPROJ = 3 * D_MODEL + IDX_HEADS * 128 + 128 + 128

NIDX = IDX_HEADS * 128 + 128 + 128


def _proj_kernel(x_ref, wqkv_ref, bqkv_ref, widx_ref, bidx_ref, y_ref, z_ref):
    x = x_ref[0]
    y = jax.lax.dot_general(x, wqkv_ref[...], (((1,), (1,)), ((), ())),
                            preferred_element_type=jnp.float32) + bqkv_ref[...]
    z = jax.lax.dot_general(x, widx_ref[...], (((1,), (1,)), ((), ())),
                            preferred_element_type=jnp.float32) + bidx_ref[...]
    s_blk = pl.program_id(1)
    t = (jax.lax.broadcasted_iota(jnp.int32, (SB1, HALF), 0).astype(jnp.float32)
         + jnp.float32(SB1) * s_blk.astype(jnp.float32))
    j = jax.lax.broadcasted_iota(jnp.int32, (SB1, HALF), 1).astype(jnp.float32)
    freqs = t * jnp.exp(j * jnp.float32(-math.log(10000.0) / HALF))
    cos = jnp.cos(freqs)
    sin = jnp.sin(freqs)
    q1 = y[:, 0:HALF]
    q2 = y[:, HALF:D_MODEL]
    k1 = y[:, D_MODEL:D_MODEL + HALF]
    k2 = y[:, D_MODEL + HALF:2 * D_MODEL]
    qr = jnp.concatenate([q1 * cos - q2 * sin, q1 * sin + q2 * cos], axis=1)
    kr = jnp.concatenate([k1 * cos - k2 * sin, k1 * sin + k2 * cos], axis=1)
    y_ref[0] = jnp.concatenate(
        [qr, kr, y[:, 2 * D_MODEL:3 * D_MODEL]], axis=1).astype(jnp.bfloat16)
    z_ref[0] = z


def _radix_select_threshold(skey, k):
    imin = jnp.int32(-2147483648)
    prefix = jnp.zeros((skey.shape[0], 1), jnp.int32)
    for bit in range(31, -1, -1):
        bp = (1 << bit) if bit < 31 else -2147483648
        cand = prefix | jnp.int32(bp)
        scand = cand ^ imin
        cnt = jnp.count_nonzero(skey >= scand, axis=1, keepdims=True)
        prefix = jnp.where(cnt >= k, cand, prefix)
    return prefix ^ imin


def _count_ge_rows(x16, scand16):
    m = jnp.where(x16 >= scand16, jnp.int16(1), jnp.int16(0))
    parts = [m[:, c * 128:(c + 1) * 128] for c in range(SEQ // 128)]
    while len(parts) > 1:
        parts = [parts[i] + parts[i + 1] for i in range(0, len(parts), 2)]
    return jnp.sum(parts[0].astype(jnp.int32), axis=1,
                   keepdims=True).astype(jnp.float32)


def _attn_kernel(qi_ref, ki_ref, wi_ref, q_ref, k_ref, v_ref, wo_ref, bo_ref,
                 tri_ref, o_ref):
    qi = qi_ref[0]
    ki = ki_ref[0]
    wi = wi_ref[0]
    agg = jnp.zeros((QB, SEQ), jnp.float32)
    for h in range(IDX_HEADS):
        sh = jax.lax.dot_general(
            qi[:, h * 128:(h + 1) * 128], ki,
            (((1,), (1,)), ((), ())), preferred_element_type=jnp.float32)
        agg = agg + jnp.maximum(sh, 0.0) * wi[:, h:h + 1]

    q = q_ref[0]
    k = k_ref[0]
    logits = jax.lax.dot_general(
        q, k, (((1,), (1,)), ((), ())),
        preferred_element_type=jnp.float32) * (1.0 / math.sqrt(D_MODEL))

    b = jax.lax.bitcast_convert_type(agg, jnp.int32)
    skey = jnp.where(b >= 0, b, b ^ jnp.int32(0x7fffffff))
    skey = jnp.where(agg == 0.0, jnp.int32(0), skey)

    sh = (skey >> 16).astype(jnp.int16)
    sl = (skey ^ jnp.int32(0x8000)).astype(jnp.int16)
    kf = jnp.float32(TOP_K)

    hpref = jnp.zeros((QB, 1), jnp.int32)
    for bit in range(15, -1, -1):
        cand = hpref | jnp.int32(1 << bit)
        scand = (cand ^ jnp.int32(0x8000)).astype(jnp.int16)
        cnt = _count_ge_rows(sh, scand)
        hpref = jnp.where(cnt >= kf, cand, hpref)
    sH = (hpref ^ jnp.int32(0x8000)).astype(jnp.int16)

    hi_gt = sh > sH
    g_cnt = _count_ge_rows(jnp.where(hi_gt, jnp.int16(1), jnp.int16(0)),
                           jnp.int16(1))
    k2f = kf - g_cnt
    active = sh == sH
    spl = jnp.where(active, sl, jnp.int16(-32768))

    lpref = jnp.zeros((QB, 1), jnp.int32)
    for bit in range(15, -1, -1):
        cand = lpref | jnp.int32(1 << bit)
        scand = (cand ^ jnp.int32(0x8000)).astype(jnp.int16)
        cnt = _count_ge_rows(spl, scand)
        lpref = jnp.where(cnt >= k2f, cand, lpref)
    sL = (lpref ^ jnp.int32(0x8000)).astype(jnp.int16)

    gt = hi_gt | (active & (spl > sL))
    eq = active & (spl == sL)
    need = kf - _count_ge_rows(jnp.where(gt, jnp.int16(1), jnp.int16(0)),
                               jnp.int16(1))
    eqf = eq.astype(jnp.float32)
    tri = tri_ref[...]
    offs = jnp.zeros((QB, 1), jnp.float32)
    parts = []
    for c in range(SEQ // 128):
        ch = eqf[:, c * 128:(c + 1) * 128]
        parts.append(jax.lax.dot_general(
            ch, tri, (((1,), (0,)), ((), ())),
            preferred_element_type=jnp.float32) + offs)
        offs = offs + jnp.sum(ch, axis=1, keepdims=True)
    cum = jnp.concatenate(parts, axis=1)
    sel = gt | (eq & (cum <= need))

    logits = jnp.where(sel, logits, -jnp.inf)
    m = jnp.max(logits, axis=1, keepdims=True)
    e = jnp.exp(logits - m)
    p = (e / jnp.sum(e, axis=1, keepdims=True)).astype(jnp.bfloat16)
    attn = jax.lax.dot_general(p, v_ref[0], (((1,), (0,)), ((), ())),
                               preferred_element_type=jnp.float32)
    out = jax.lax.dot_general(attn.astype(jnp.bfloat16),
                              wo_ref[...].astype(jnp.bfloat16),
                              (((1,), (1,)), ((), ())),
                              preferred_element_type=jnp.float32) + bo_ref[...]
    o_ref[0] = out


@jax.jit
def kernel(x, Wqkv, bqkv, Wq_idx, bq_idx, Wk_idx, bk_idx, Ww_idx, bw_idx, Wo,
           bo):
    B, S, D = x.shape

    w_idx = jnp.zeros((NIDX, D), jnp.float32)
    b_idx = jnp.zeros((NIDX,), jnp.float32)
    for h in range(IDX_HEADS):
        w_idx = jax.lax.dynamic_update_slice(
            w_idx, Wq_idx[h * IDX_DIM:(h + 1) * IDX_DIM], (h * 128, 0))
        b_idx = jax.lax.dynamic_update_slice(
            b_idx, bq_idx[h * IDX_DIM:(h + 1) * IDX_DIM], (h * 128,))
    w_idx = jax.lax.dynamic_update_slice(w_idx, Wk_idx, (IDX_HEADS * 128, 0))
    b_idx = jax.lax.dynamic_update_slice(b_idx, bk_idx, (IDX_HEADS * 128,))
    w_idx = jax.lax.dynamic_update_slice(w_idx, Ww_idx, (IDX_HEADS * 128 + 128, 0))
    b_idx = jax.lax.dynamic_update_slice(b_idx, bw_idx, (IDX_HEADS * 128 + 128,))

    ii = jnp.arange(128, dtype=jnp.int32)
    tri = (ii[:, None] <= ii[None, :]).astype(jnp.float32)

    y, z = pl.pallas_call(
        _proj_kernel,
        grid=(B, S // SB1),
        in_specs=[
            pl.BlockSpec((1, SB1, D), lambda b, s: (b, s, 0)),
            pl.BlockSpec((3 * D_MODEL, D), lambda b, s: (0, 0)),
            pl.BlockSpec((1, 3 * D_MODEL), lambda b, s: (0, 0)),
            pl.BlockSpec((NIDX, D), lambda b, s: (0, 0)),
            pl.BlockSpec((1, NIDX), lambda b, s: (0, 0)),
        ],
        out_specs=[
            pl.BlockSpec((1, SB1, 3 * D_MODEL), lambda b, s: (b, s, 0)),
            pl.BlockSpec((1, SB1, NIDX), lambda b, s: (b, s, 0)),
        ],
        out_shape=[
            jax.ShapeDtypeStruct((B, S, 3 * D_MODEL), jnp.bfloat16),
            jax.ShapeDtypeStruct((B, S, NIDX), jnp.float32),
        ],
    )(x, Wqkv, bqkv[None, :], w_idx, b_idx[None, :])

    out = pl.pallas_call(
        _attn_kernel,
        grid=(B, S // QB),
        in_specs=[
            pl.BlockSpec((1, QB, IDX_HEADS * 128), lambda b, q: (b, q, 0)),
            pl.BlockSpec((1, SEQ, 128), lambda b, q: (b, 0, 4)),
            pl.BlockSpec((1, QB, 128), lambda b, q: (b, q, 5)),
            pl.BlockSpec((1, QB, D_MODEL), lambda b, q: (b, q, 0)),
            pl.BlockSpec((1, SEQ, D_MODEL), lambda b, q: (b, 0, 1)),
            pl.BlockSpec((1, SEQ, D_MODEL), lambda b, q: (b, 0, 2)),
            pl.BlockSpec((D_MODEL, D_MODEL), lambda b, q: (0, 0)),
            pl.BlockSpec((1, D_MODEL), lambda b, q: (0, 0)),
            pl.BlockSpec((128, 128), lambda b, q: (0, 0)),
        ],
        out_specs=pl.BlockSpec((1, QB, D_MODEL), lambda b, q: (b, q, 0)),
        out_shape=jax.ShapeDtypeStruct((B, S, D_MODEL), jnp.float32),
    )(z, z, z, y, y, y, Wo, bo[None, :], tri)
    return out
